# baseline (speedup 1.0000x reference)
import functools
import math

import jax
import jax.numpy as jnp
from jax.experimental import pallas as pl
from jax.experimental.pallas import tpu as pltpu

_M = 0.2
_S = 30.0
_COS_M = math.cos(_M)
_SIN_M = math.sin(_M)
_TH = math.cos(math.pi - _M)
_MM = math.sin(math.pi - _M) * _M

_BR = 256
_BC = 2048


def _body(lab_ref, cos_ref, out_ref):
    j = pl.program_id(1)
    x = cos_ref[...]
    lab = lab_ref[...]
    sine = jnp.sqrt(jnp.clip(1.0 - x * x, 0.0, 1.0))
    phi = x * _COS_M - sine * _SIN_M
    phi = jnp.where(x - _TH > 0, phi, x - _MM)
    col = j * _BC + jax.lax.broadcasted_iota(jnp.int32, x.shape, 1)
    mask = lab == col
    out_ref[...] = _S * jnp.where(mask, phi, x)


def kernel(cosine, label):
    n, v = cosine.shape
    lab2d = label.astype(jnp.int32).reshape(n, 1)
    grid = (n // _BR, pl.cdiv(v, _BC))
    return pl.pallas_call(
        _body,
        grid=grid,
        in_specs=[
            pl.BlockSpec((_BR, 1), lambda i, j: (i, 0)),
            pl.BlockSpec((_BR, _BC), lambda i, j: (i, j)),
        ],
        out_specs=pl.BlockSpec((_BR, _BC), lambda i, j: (i, j)),
        out_shape=jax.ShapeDtypeStruct((n, v), jnp.float32),
        compiler_params=pltpu.CompilerParams(
            dimension_semantics=("parallel", "parallel"),
        ),
    )(lab2d, cosine)
